# Initial kernel scaffold; baseline (speedup 1.0000x reference)
import jax
import jax.numpy as jnp
from jax.experimental import pallas as pl


def kernel(x):
    raise NotImplementedError("write your pallas kernel here")


# speedup vs baseline: 5.0896x; 5.0896x over previous
import functools

import jax
import jax.numpy as jnp
from jax import lax
from jax.experimental import pallas as pl
from jax.experimental.pallas import tpu as pltpu
from jax.experimental.pallas import tpu_sc as plsc

B = 8
N = 4096
FEAT = 6
NPOINT = 512
K = 16
SL, LN = 32, 128
FPAD = 8

BIGF = 3.0e38


def _fps_knn_kernel(xyz_ref, knn_ref):
    x0 = xyz_ref[0]
    x1 = xyz_ref[1]
    x2 = xyz_ref[2]
    jidx = (lax.broadcasted_iota(jnp.int32, (B, SL, LN), 1) * LN
            + lax.broadcasted_iota(jnp.int32, (B, SL, LN), 2))
    boff = lax.broadcasted_iota(jnp.int32, (B, 1, 1), 0) * N

    def body(i, carry):
        distance, farthest = carry
        sel = jidx == farthest
        cx = jnp.sum(jnp.where(sel, x0, 0.0), axis=(1, 2), keepdims=True)
        cy = jnp.sum(jnp.where(sel, x1, 0.0), axis=(1, 2), keepdims=True)
        cz = jnp.sum(jnp.where(sel, x2, 0.0), axis=(1, 2), keepdims=True)
        dx = x0 - cx
        dy = x1 - cy
        dz = x2 - cz
        dist = dx * dx + dy * dy + dz * dz
        distance = jnp.minimum(distance, dist)
        m = jnp.max(distance, axis=(1, 2), keepdims=True)
        farthest = jnp.min(jnp.where(distance == m, jidx, N),
                           axis=(1, 2), keepdims=True)
        d = dist
        cols = []
        for _ in range(K):
            mn = jnp.min(d, axis=(1, 2), keepdims=True)
            t = jnp.where(d == mn, jidx, N)
            ji = jnp.min(t, axis=(1, 2), keepdims=True)
            d = jnp.where(jidx == ji, BIGF, d)
            cols.append(ji + boff)
        row = jnp.concatenate(cols, axis=2)
        knn_ref[pl.ds(i, 1)] = row.reshape(1, B, K)
        return distance, farthest

    distance0 = jnp.full((B, SL, LN), 1e10, dtype=jnp.float32)
    farthest0 = jnp.zeros((B, 1, 1), dtype=jnp.int32)
    lax.fori_loop(0, NPOINT, body, (distance0, farthest0))


def _fps_knn(xyzT):
    return pl.pallas_call(
        _fps_knn_kernel,
        out_shape=jax.ShapeDtypeStruct((NPOINT, B, K), jnp.int32),
    )(xyzT)


def _sc_gather(table, idx):
    info = plsc.get_sparse_core_info()
    nc, ns = info.num_cores, info.num_subcores
    nw = nc * ns
    total = B * NPOINT * K
    ch = total // (nw * 128)

    mesh = plsc.VectorSubcoreMesh(core_axis_name="c", subcore_axis_name="s")

    @functools.partial(
        pl.kernel,
        mesh=mesh,
        out_type=jax.ShapeDtypeStruct((nw, ch, 128, FPAD), jnp.float32),
        scratch_types=[
            pltpu.VMEM((ch, 128), jnp.int32),
            pltpu.VMEM((ch, 128, FPAD), jnp.float32),
            pltpu.SemaphoreType.DMA,
        ],
        compiler_params=pltpu.CompilerParams(use_tc_tiling_on_sc=False),
    )
    def gather_k(tab_hbm, idx_hbm, out_hbm, idx_v, rows_v, sem):
        w = lax.axis_index("s") * nc + lax.axis_index("c")
        pltpu.sync_copy(idx_hbm.at[w], idx_v)
        copies = []
        for c in range(ch):
            copies.append(
                pltpu.async_copy(tab_hbm.at[idx_v.at[c]], rows_v.at[c], sem))
        for cp in copies:
            cp.wait()
        pltpu.sync_copy(rows_v, out_hbm.at[w])

    return gather_k(table, idx)


def kernel(x):
    xyzT = x[:, :, :3].transpose(2, 0, 1).reshape(3, B, SL, LN)
    knn = _fps_knn(xyzT)

    info = plsc.get_sparse_core_info()
    nw = info.num_cores * info.num_subcores
    total = B * NPOINT * K
    ch = total // (nw * 128)
    idx = knn.transpose(1, 0, 2).reshape(nw, ch, 128)

    table = jnp.pad(x, ((0, 0), (0, 0), (0, FPAD - FEAT))).reshape(B * N, FPAD)
    rows = _sc_gather(table, idx)
    return rows.reshape(B, NPOINT, K, FPAD)[..., :FEAT]


# speedup vs baseline: 7.1597x; 1.4067x over previous
import functools

import jax
import jax.numpy as jnp
from jax import lax
from jax.experimental import pallas as pl
from jax.experimental.pallas import tpu as pltpu
from jax.experimental.pallas import tpu_sc as plsc

B = 8
N = 4096
FEAT = 6
NPOINT = 512
K = 16
SL, LN = 32, 128
FPAD = 8

BIGF = 3.0e38


def _fps_knn_kernel(xyz_ref, knn_ref):
    x0 = xyz_ref[0]
    x1 = xyz_ref[1]
    x2 = xyz_ref[2]
    jidx = (lax.broadcasted_iota(jnp.int32, (SL, B, LN), 0) * LN
            + lax.broadcasted_iota(jnp.int32, (SL, B, LN), 2)
            ).astype(jnp.float32)
    boff = (lax.broadcasted_iota(jnp.int32, (1, B, 1), 1) * N
            ).astype(jnp.float32)

    def extract_row(d):
        cols = []
        for _ in range(K):
            mn = jnp.min(d, axis=(0, 2), keepdims=True)
            t = jnp.where(d == mn, jidx, float(N))
            ji = jnp.min(t, axis=(0, 2), keepdims=True)
            d = jnp.where(t == ji, BIGF, d)
            cols.append((ji + boff).astype(jnp.int32))
        return jnp.concatenate(cols, axis=2)

    def fps_step(distance, farthest):
        sel = jidx == farthest
        cx = jnp.sum(jnp.where(sel, x0, 0.0), axis=(0, 2), keepdims=True)
        cy = jnp.sum(jnp.where(sel, x1, 0.0), axis=(0, 2), keepdims=True)
        cz = jnp.sum(jnp.where(sel, x2, 0.0), axis=(0, 2), keepdims=True)
        dx = x0 - cx
        dy = x1 - cy
        dz = x2 - cz
        dist = dx * dx + dy * dy + dz * dz
        distance = jnp.minimum(distance, dist)
        m = jnp.max(distance, axis=(0, 2), keepdims=True)
        farthest = jnp.min(jnp.where(distance == m, jidx, float(N)),
                           axis=(0, 2), keepdims=True)
        return dist, distance, farthest

    def body(i, carry):
        distance, farthest, dist_prev = carry
        knn_ref[pl.ds(jnp.maximum(i - 1, 0), 1)] = extract_row(dist_prev)
        dist, distance, farthest = fps_step(distance, farthest)
        return distance, farthest, dist

    distance0 = jnp.full((SL, B, LN), 1e10, dtype=jnp.float32)
    farthest0 = jnp.zeros((1, B, 1), dtype=jnp.float32)
    dist0 = jnp.zeros((SL, B, LN), dtype=jnp.float32)
    _, _, dist_last = lax.fori_loop(0, NPOINT, body,
                                    (distance0, farthest0, dist0))
    knn_ref[pl.ds(NPOINT - 1, 1)] = extract_row(dist_last)


def _fps_knn(xyzT):
    return pl.pallas_call(
        _fps_knn_kernel,
        out_shape=jax.ShapeDtypeStruct((NPOINT, B, K), jnp.int32),
    )(xyzT)


def _sc_gather(table, idx):
    info = plsc.get_sparse_core_info()
    nc, ns = info.num_cores, info.num_subcores
    nw = nc * ns
    total = B * NPOINT * K
    ch = total // (nw * 128)

    mesh = plsc.VectorSubcoreMesh(core_axis_name="c", subcore_axis_name="s")

    @functools.partial(
        pl.kernel,
        mesh=mesh,
        out_type=jax.ShapeDtypeStruct((nw, ch, 128, FPAD), jnp.float32),
        scratch_types=[
            pltpu.VMEM((ch, 128), jnp.int32),
            pltpu.VMEM((ch, 128, FPAD), jnp.float32),
            pltpu.SemaphoreType.DMA,
        ],
        compiler_params=pltpu.CompilerParams(use_tc_tiling_on_sc=False),
    )
    def gather_k(tab_hbm, idx_hbm, out_hbm, idx_v, rows_v, sem):
        w = lax.axis_index("s") * nc + lax.axis_index("c")
        pltpu.sync_copy(idx_hbm.at[w], idx_v)
        copies = []
        for c in range(ch):
            copies.append(
                pltpu.async_copy(tab_hbm.at[idx_v.at[c]], rows_v.at[c], sem))
        for cp in copies:
            cp.wait()
        pltpu.sync_copy(rows_v, out_hbm.at[w])

    return gather_k(table, idx)


def kernel(x):
    xyzT = (x[:, :, :3].transpose(2, 0, 1).reshape(3, B, SL, LN)
            .transpose(0, 2, 1, 3))
    knn = _fps_knn(xyzT)

    info = plsc.get_sparse_core_info()
    nw = info.num_cores * info.num_subcores
    total = B * NPOINT * K
    ch = total // (nw * 128)
    idx = knn.transpose(1, 0, 2).reshape(nw, ch, 128)

    table = jnp.pad(x, ((0, 0), (0, 0), (0, FPAD - FEAT))).reshape(B * N, FPAD)
    rows = _sc_gather(table, idx)
    return rows.reshape(B, NPOINT, K, FPAD)[..., :FEAT]


# speedup vs baseline: 20.1116x; 2.8090x over previous
import functools

import jax
import jax.numpy as jnp
from jax import lax
from jax.experimental import pallas as pl
from jax.experimental.pallas import tpu as pltpu
from jax.experimental.pallas import tpu_sc as plsc

B = 8
N = 4096
FEAT = 6
NPOINT = 512
K = 16
SL, LN = 32, 128
FPAD = 8

BIGF = 3.0e38


TR = 16


def _jidx():
    return (lax.broadcasted_iota(jnp.int32, (SL, B, LN), 0) * LN
            + lax.broadcasted_iota(jnp.int32, (SL, B, LN), 2)
            ).astype(jnp.float32)


def _fps_kernel(xyz_ref, dist_ref, distance_s, far_s):
    i = pl.program_id(0)

    @pl.when(i == 0)
    def _init():
        distance_s[...] = jnp.full((SL, B, LN), 1e10, dtype=jnp.float32)
        far_s[...] = jnp.zeros((1, B, LN), dtype=jnp.float32)

    x0 = xyz_ref[0]
    x1 = xyz_ref[1]
    x2 = xyz_ref[2]
    jidx = _jidx()
    farthest = far_s[:, :, 0:1]
    sel = jidx == farthest
    cx = jnp.sum(jnp.where(sel, x0, 0.0), axis=(0, 2), keepdims=True)
    cy = jnp.sum(jnp.where(sel, x1, 0.0), axis=(0, 2), keepdims=True)
    cz = jnp.sum(jnp.where(sel, x2, 0.0), axis=(0, 2), keepdims=True)
    dx = x0 - cx
    dy = x1 - cy
    dz = x2 - cz
    dist = dx * dx + dy * dy + dz * dz
    distance = jnp.minimum(distance_s[...], dist)
    m = jnp.max(distance, axis=(0, 2), keepdims=True)
    nxt = jnp.min(jnp.where(distance == m, jidx, float(N)),
                  axis=(0, 2), keepdims=True)
    dist_ref[0] = dist
    distance_s[...] = distance
    far_s[...] = jnp.broadcast_to(nxt, (1, B, LN))


def _extract_kernel(dist_ref, knn_ref):
    jidx = _jidx()
    boff = (lax.broadcasted_iota(jnp.int32, (1, B, 1), 1) * N
            ).astype(jnp.float32)
    for r in range(TR):
        d = dist_ref[r]
        cols = []
        for _ in range(K):
            mn = jnp.min(d, axis=(0, 2), keepdims=True)
            t = jnp.where(d == mn, jidx, float(N))
            ji = jnp.min(t, axis=(0, 2), keepdims=True)
            d = jnp.where(t == ji, BIGF, d)
            cols.append((ji + boff).astype(jnp.int32))
        knn_ref[pl.ds(r, 1)] = jnp.concatenate(cols, axis=2)


def _fps_knn(xyzT):
    dist_rows = pl.pallas_call(
        _fps_kernel,
        grid=(NPOINT,),
        in_specs=[pl.BlockSpec((3, SL, B, LN), lambda i: (0, 0, 0, 0))],
        out_specs=pl.BlockSpec((1, SL, B, LN), lambda i: (i, 0, 0, 0)),
        out_shape=jax.ShapeDtypeStruct((NPOINT, SL, B, LN), jnp.float32),
        scratch_shapes=[
            pltpu.VMEM((SL, B, LN), jnp.float32),
            pltpu.VMEM((1, B, LN), jnp.float32),
        ],
    )(xyzT)
    return pl.pallas_call(
        _extract_kernel,
        grid=(NPOINT // TR,),
        in_specs=[pl.BlockSpec((TR, SL, B, LN), lambda i: (i, 0, 0, 0))],
        out_specs=pl.BlockSpec((TR, B, K), lambda i: (i, 0, 0)),
        out_shape=jax.ShapeDtypeStruct((NPOINT, B, K), jnp.int32),
    )(dist_rows)


def _sc_gather(table, idx):
    info = plsc.get_sparse_core_info()
    nc, ns = info.num_cores, info.num_subcores
    nw = nc * ns
    total = B * NPOINT * K
    ch = total // (nw * 128)

    mesh = plsc.VectorSubcoreMesh(core_axis_name="c", subcore_axis_name="s")

    @functools.partial(
        pl.kernel,
        mesh=mesh,
        out_type=jax.ShapeDtypeStruct((nw, ch, 128, FPAD), jnp.float32),
        scratch_types=[
            pltpu.VMEM((ch, 128), jnp.int32),
            pltpu.VMEM((ch, 128, FPAD), jnp.float32),
            pltpu.SemaphoreType.DMA,
        ],
        compiler_params=pltpu.CompilerParams(use_tc_tiling_on_sc=False),
    )
    def gather_k(tab_hbm, idx_hbm, out_hbm, idx_v, rows_v, sem):
        w = lax.axis_index("s") * nc + lax.axis_index("c")
        pltpu.sync_copy(idx_hbm.at[w], idx_v)
        copies = []
        for c in range(ch):
            copies.append(
                pltpu.async_copy(tab_hbm.at[idx_v.at[c]], rows_v.at[c], sem))
        for cp in copies:
            cp.wait()
        pltpu.sync_copy(rows_v, out_hbm.at[w])

    return gather_k(table, idx)


def kernel(x):
    xyzT = (x[:, :, :3].transpose(2, 0, 1).reshape(3, B, SL, LN)
            .transpose(0, 2, 1, 3))
    knn = _fps_knn(xyzT)

    info = plsc.get_sparse_core_info()
    nw = info.num_cores * info.num_subcores
    total = B * NPOINT * K
    ch = total // (nw * 128)
    idx = knn.transpose(1, 0, 2).reshape(nw, ch, 128)

    table = jnp.pad(x, ((0, 0), (0, 0), (0, FPAD - FEAT))).reshape(B * N, FPAD)
    rows = _sc_gather(table, idx)
    return rows.reshape(B, NPOINT, K, FPAD)[..., :FEAT]


# speedup vs baseline: 20.5912x; 1.0238x over previous
import functools

import jax
import jax.numpy as jnp
from jax import lax
from jax.experimental import pallas as pl
from jax.experimental.pallas import tpu as pltpu
from jax.experimental.pallas import tpu_sc as plsc

B = 8
N = 4096
FEAT = 6
NPOINT = 512
K = 16
SL, LN = 32, 128
FPAD = 8

BIGF = 3.0e38


TR = 16


def _fps_kernel(xyzj_ref, dist_ref, distance_s, cen_s):
    i = pl.program_id(0)
    x0 = xyzj_ref[0]
    x1 = xyzj_ref[1]
    x2 = xyzj_ref[2]
    jidx = xyzj_ref[3]

    @pl.when(i == 0)
    def _init():
        distance_s[...] = jnp.full((SL, B, LN), 1e10, dtype=jnp.float32)
        cen_s[0:1] = x0[0:1, :, 0:1]
        cen_s[1:2] = x1[0:1, :, 0:1]
        cen_s[2:3] = x2[0:1, :, 0:1]

    cx = cen_s[0:1]
    cy = cen_s[1:2]
    cz = cen_s[2:3]
    dx = x0 - cx
    dy = x1 - cy
    dz = x2 - cz
    dist = dx * dx + dy * dy + dz * dz
    distance = jnp.minimum(distance_s[...], dist)
    dist_ref[0] = dist
    distance_s[...] = distance

    d, px, py, pz, pj = distance, x0, x1, x2, jidx
    n = SL
    while n > 1:
        h = n // 2
        cond = d[:h] >= d[h:n]
        d = jnp.where(cond, d[:h], d[h:n])
        px = jnp.where(cond, px[:h], px[h:n])
        py = jnp.where(cond, py[:h], py[h:n])
        pz = jnp.where(cond, pz[:h], pz[h:n])
        pj = jnp.where(cond, pj[:h], pj[h:n])
        n = h
    mx = jnp.max(d, axis=2, keepdims=True)
    mask = d == mx
    jm = jnp.min(jnp.where(mask, pj, BIGF), axis=2, keepdims=True)
    s2 = pj == jm
    cen_s[0:1] = jnp.sum(jnp.where(s2, px, 0.0), axis=2, keepdims=True)
    cen_s[1:2] = jnp.sum(jnp.where(s2, py, 0.0), axis=2, keepdims=True)
    cen_s[2:3] = jnp.sum(jnp.where(s2, pz, 0.0), axis=2, keepdims=True)


def _extract_kernel(dist_ref, jidx_ref, knn_ref):
    jidx = jidx_ref[0]
    boff = (lax.broadcasted_iota(jnp.int32, (1, B, 1), 1) * N
            ).astype(jnp.float32)
    for r in range(TR):
        d = dist_ref[r]
        cols = []
        for _ in range(K):
            mn = jnp.min(d, axis=(0, 2), keepdims=True)
            t = jnp.where(d == mn, jidx, float(N))
            ji = jnp.min(t, axis=(0, 2), keepdims=True)
            d = jnp.where(t == ji, BIGF, d)
            cols.append((ji + boff).astype(jnp.int32))
        knn_ref[pl.ds(r, 1)] = jnp.concatenate(cols, axis=2)


def _fps_knn(xyzT):
    jplane = (jnp.arange(SL * LN, dtype=jnp.int32).astype(jnp.float32)
              .reshape(SL, 1, LN))
    xyzj = jnp.concatenate(
        [xyzT, jnp.broadcast_to(jplane, (1, SL, B, LN))], axis=0)
    dist_rows = pl.pallas_call(
        _fps_kernel,
        grid=(NPOINT,),
        in_specs=[pl.BlockSpec((4, SL, B, LN), lambda i: (0, 0, 0, 0))],
        out_specs=pl.BlockSpec((1, SL, B, LN), lambda i: (i, 0, 0, 0)),
        out_shape=jax.ShapeDtypeStruct((NPOINT, SL, B, LN), jnp.float32),
        scratch_shapes=[
            pltpu.VMEM((SL, B, LN), jnp.float32),
            pltpu.VMEM((3, B, 1), jnp.float32),
        ],
    )(xyzj)
    return pl.pallas_call(
        _extract_kernel,
        grid=(NPOINT // TR,),
        in_specs=[
            pl.BlockSpec((TR, SL, B, LN), lambda i: (i, 0, 0, 0)),
            pl.BlockSpec((1, SL, B, LN), lambda i: (0, 0, 0, 0)),
        ],
        out_specs=pl.BlockSpec((TR, B, K), lambda i: (i, 0, 0)),
        out_shape=jax.ShapeDtypeStruct((NPOINT, B, K), jnp.int32),
        compiler_params=pltpu.CompilerParams(
            dimension_semantics=("parallel",)),
    )(dist_rows, xyzj[3:4])


def _sc_gather(table, idx):
    info = plsc.get_sparse_core_info()
    nc, ns = info.num_cores, info.num_subcores
    nw = nc * ns
    total = B * NPOINT * K
    ch = total // (nw * 128)

    mesh = plsc.VectorSubcoreMesh(core_axis_name="c", subcore_axis_name="s")

    @functools.partial(
        pl.kernel,
        mesh=mesh,
        out_type=jax.ShapeDtypeStruct((nw, ch, 128, FPAD), jnp.float32),
        scratch_types=[
            pltpu.VMEM((ch, 128), jnp.int32),
            pltpu.VMEM((ch, 128, FPAD), jnp.float32),
            pltpu.SemaphoreType.DMA,
        ],
        compiler_params=pltpu.CompilerParams(use_tc_tiling_on_sc=False),
    )
    def gather_k(tab_hbm, idx_hbm, out_hbm, idx_v, rows_v, sem):
        w = lax.axis_index("s") * nc + lax.axis_index("c")
        pltpu.sync_copy(idx_hbm.at[w], idx_v)
        copies = []
        for c in range(ch):
            copies.append(
                pltpu.async_copy(tab_hbm.at[idx_v.at[c]], rows_v.at[c], sem))
        for cp in copies:
            cp.wait()
        pltpu.sync_copy(rows_v, out_hbm.at[w])

    return gather_k(table, idx)


def kernel(x):
    xyzT = (x[:, :, :3].transpose(2, 0, 1).reshape(3, B, SL, LN)
            .transpose(0, 2, 1, 3))
    knn = _fps_knn(xyzT)

    info = plsc.get_sparse_core_info()
    nw = info.num_cores * info.num_subcores
    total = B * NPOINT * K
    ch = total // (nw * 128)
    idx = knn.transpose(1, 0, 2).reshape(nw, ch, 128)

    table = jnp.pad(x, ((0, 0), (0, 0), (0, FPAD - FEAT))).reshape(B * N, FPAD)
    rows = _sc_gather(table, idx)
    return rows.reshape(B, NPOINT, K, FPAD)[..., :FEAT]


# speedup vs baseline: 22.4800x; 1.0917x over previous
import functools

import jax
import jax.numpy as jnp
from jax import lax
from jax.experimental import pallas as pl
from jax.experimental.pallas import tpu as pltpu
from jax.experimental.pallas import tpu_sc as plsc

B = 8
N = 4096
FEAT = 6
NPOINT = 512
K = 16
SL, LN = 32, 128
FPAD = 8

BIGF = 3.0e38


TR = 16


FPI = 8


def _fps_kernel(xyzj_ref, dist_ref, distance_s, cen_s):
    i = pl.program_id(0)
    x0 = xyzj_ref[0]
    x1 = xyzj_ref[1]
    x2 = xyzj_ref[2]
    jidx = xyzj_ref[3]

    @pl.when(i == 0)
    def _init():
        distance_s[...] = jnp.full((SL, B, LN), 1e10, dtype=jnp.float32)
        cen_s[0:1] = x0[0:1, :, 0:1]
        cen_s[1:2] = x1[0:1, :, 0:1]
        cen_s[2:3] = x2[0:1, :, 0:1]

    cx = cen_s[0:1]
    cy = cen_s[1:2]
    cz = cen_s[2:3]
    distance = distance_s[...]
    for s in range(FPI):
        dx = x0 - cx
        dy = x1 - cy
        dz = x2 - cz
        dist = dx * dx + dy * dy + dz * dz
        distance = jnp.minimum(distance, dist)
        dist_ref[s] = dist
        d, px, py, pz, pj = distance, x0, x1, x2, jidx
        n = SL
        while n > 1:
            h = n // 2
            cond = d[:h] >= d[h:n]
            d = jnp.where(cond, d[:h], d[h:n])
            px = jnp.where(cond, px[:h], px[h:n])
            py = jnp.where(cond, py[:h], py[h:n])
            pz = jnp.where(cond, pz[:h], pz[h:n])
            pj = jnp.where(cond, pj[:h], pj[h:n])
            n = h
        mx = jnp.max(d, axis=2, keepdims=True)
        jm = jnp.min(jnp.where(d == mx, pj, BIGF), axis=2, keepdims=True)
        s2 = pj == jm
        cx = jnp.sum(jnp.where(s2, px, 0.0), axis=2, keepdims=True)
        cy = jnp.sum(jnp.where(s2, py, 0.0), axis=2, keepdims=True)
        cz = jnp.sum(jnp.where(s2, pz, 0.0), axis=2, keepdims=True)
    distance_s[...] = distance
    cen_s[0:1] = cx
    cen_s[1:2] = cy
    cen_s[2:3] = cz


def _extract_kernel(dist_ref, jidx_ref, knn_ref):
    jidx = jidx_ref[0]
    boff = (lax.broadcasted_iota(jnp.int32, (1, B, 1), 1) * N
            ).astype(jnp.float32)
    for r in range(TR):
        d = dist_ref[r]
        cols = []
        for _ in range(K):
            mn = jnp.min(d, axis=(0, 2), keepdims=True)
            t = jnp.where(d == mn, jidx, float(N))
            ji = jnp.min(t, axis=(0, 2), keepdims=True)
            d = jnp.where(t == ji, BIGF, d)
            cols.append((ji + boff).astype(jnp.int32))
        knn_ref[pl.ds(r, 1)] = jnp.concatenate(cols, axis=2)


def _fps_knn(xyzT):
    jplane = (jnp.arange(SL * LN, dtype=jnp.int32).astype(jnp.float32)
              .reshape(SL, 1, LN))
    xyzj = jnp.concatenate(
        [xyzT, jnp.broadcast_to(jplane, (1, SL, B, LN))], axis=0)
    dist_rows = pl.pallas_call(
        _fps_kernel,
        grid=(NPOINT // FPI,),
        in_specs=[pl.BlockSpec((4, SL, B, LN), lambda i: (0, 0, 0, 0))],
        out_specs=pl.BlockSpec((FPI, SL, B, LN), lambda i: (i, 0, 0, 0)),
        out_shape=jax.ShapeDtypeStruct((NPOINT, SL, B, LN), jnp.float32),
        scratch_shapes=[
            pltpu.VMEM((SL, B, LN), jnp.float32),
            pltpu.VMEM((3, B, 1), jnp.float32),
        ],
    )(xyzj)
    return pl.pallas_call(
        _extract_kernel,
        grid=(NPOINT // TR,),
        in_specs=[
            pl.BlockSpec((TR, SL, B, LN), lambda i: (i, 0, 0, 0)),
            pl.BlockSpec((1, SL, B, LN), lambda i: (0, 0, 0, 0)),
        ],
        out_specs=pl.BlockSpec((TR, B, K), lambda i: (i, 0, 0)),
        out_shape=jax.ShapeDtypeStruct((NPOINT, B, K), jnp.int32),
        compiler_params=pltpu.CompilerParams(
            dimension_semantics=("parallel",)),
    )(dist_rows, xyzj[3:4])


def _sc_gather(table, idx):
    info = plsc.get_sparse_core_info()
    nc, ns = info.num_cores, info.num_subcores
    nw = nc * ns
    total = B * NPOINT * K
    ch = total // (nw * 128)

    mesh = plsc.VectorSubcoreMesh(core_axis_name="c", subcore_axis_name="s")

    @functools.partial(
        pl.kernel,
        mesh=mesh,
        out_type=jax.ShapeDtypeStruct((nw, ch, 128, FPAD), jnp.float32),
        scratch_types=[
            pltpu.VMEM((ch, 128), jnp.int32),
            pltpu.VMEM((ch, 128, FPAD), jnp.float32),
            pltpu.SemaphoreType.DMA,
        ],
        compiler_params=pltpu.CompilerParams(use_tc_tiling_on_sc=False),
    )
    def gather_k(tab_hbm, idx_hbm, out_hbm, idx_v, rows_v, sem):
        w = lax.axis_index("s") * nc + lax.axis_index("c")
        pltpu.sync_copy(idx_hbm.at[w], idx_v)
        copies = []
        for c in range(ch):
            copies.append(
                pltpu.async_copy(tab_hbm.at[idx_v.at[c]], rows_v.at[c], sem))
        for cp in copies:
            cp.wait()
        pltpu.sync_copy(rows_v, out_hbm.at[w])

    return gather_k(table, idx)


def kernel(x):
    xyzT = (x[:, :, :3].transpose(2, 0, 1).reshape(3, B, SL, LN)
            .transpose(0, 2, 1, 3))
    knn = _fps_knn(xyzT)

    info = plsc.get_sparse_core_info()
    nw = info.num_cores * info.num_subcores
    total = B * NPOINT * K
    ch = total // (nw * 128)
    idx = knn.transpose(1, 0, 2).reshape(nw, ch, 128)

    table = jnp.pad(x, ((0, 0), (0, 0), (0, FPAD - FEAT))).reshape(B * N, FPAD)
    rows = _sc_gather(table, idx)
    return rows.reshape(B, NPOINT, K, FPAD)[..., :FEAT]


# speedup vs baseline: 28.0827x; 1.2492x over previous
import functools

import jax
import jax.numpy as jnp
from jax import lax
from jax.experimental import pallas as pl
from jax.experimental.pallas import tpu as pltpu
from jax.experimental.pallas import tpu_sc as plsc

B = 8
N = 4096
FEAT = 6
NPOINT = 512
K = 16
SL, LN = 32, 128
FPAD = 8

BIGF = 3.0e38


TR = 16


FPI = 8


def _fps_knn_fused(xyzj_ref, knn_ref, distance_s, cen_s, ring_s):
    i = pl.program_id(0)
    x0 = xyzj_ref[0]
    x1 = xyzj_ref[1]
    x2 = xyzj_ref[2]
    jidx = xyzj_ref[3]

    @pl.when(i == 0)
    def _init():
        distance_s[...] = jnp.full((SL, B, LN), 1e10, dtype=jnp.float32)
        cen_s[0:1] = x0[0:1, :, 0:1]
        cen_s[1:2] = x1[0:1, :, 0:1]
        cen_s[2:3] = x2[0:1, :, 0:1]
        ring_s[...] = jnp.zeros(ring_s.shape, dtype=jnp.float32)

    wslot = lax.rem(i, 2)
    rslot = lax.rem(i + 1, 2)

    boff = (lax.broadcasted_iota(jnp.int32, (1, B, 1), 1) * N
            ).astype(jnp.float32)
    for r in range(FPI):
        d = ring_s[rslot, r]
        cols = []
        for _ in range(K):
            mn = jnp.min(d, axis=(0, 2), keepdims=True)
            t = jnp.where(d == mn, jidx, float(N))
            ji = jnp.min(t, axis=(0, 2), keepdims=True)
            d = jnp.where(t == ji, BIGF, d)
            cols.append((ji + boff).astype(jnp.int32))
        knn_ref[pl.ds(r, 1)] = jnp.concatenate(cols, axis=2)

    cx = cen_s[0:1]
    cy = cen_s[1:2]
    cz = cen_s[2:3]
    distance = distance_s[...]
    for s in range(FPI):
        dx = x0 - cx
        dy = x1 - cy
        dz = x2 - cz
        dist = dx * dx + dy * dy + dz * dz
        distance = jnp.minimum(distance, dist)
        ring_s[wslot, s] = dist
        d, px, py, pz, pj = distance, x0, x1, x2, jidx
        n = SL
        while n > 1:
            h = n // 2
            cond = d[:h] >= d[h:n]
            d = jnp.where(cond, d[:h], d[h:n])
            px = jnp.where(cond, px[:h], px[h:n])
            py = jnp.where(cond, py[:h], py[h:n])
            pz = jnp.where(cond, pz[:h], pz[h:n])
            pj = jnp.where(cond, pj[:h], pj[h:n])
            n = h
        mx = jnp.max(d, axis=2, keepdims=True)
        jm = jnp.min(jnp.where(d == mx, pj, BIGF), axis=2, keepdims=True)
        s2 = pj == jm
        cx = jnp.sum(jnp.where(s2, px, 0.0), axis=2, keepdims=True)
        cy = jnp.sum(jnp.where(s2, py, 0.0), axis=2, keepdims=True)
        cz = jnp.sum(jnp.where(s2, pz, 0.0), axis=2, keepdims=True)
    distance_s[...] = distance
    cen_s[0:1] = cx
    cen_s[1:2] = cy
    cen_s[2:3] = cz


def _fps_knn(xyzT):
    jplane = (jnp.arange(SL * LN, dtype=jnp.int32).astype(jnp.float32)
              .reshape(SL, 1, LN))
    xyzj = jnp.concatenate(
        [xyzT, jnp.broadcast_to(jplane, (1, SL, B, LN))], axis=0)
    nstep = NPOINT // FPI
    return pl.pallas_call(
        _fps_knn_fused,
        grid=(nstep + 1,),
        in_specs=[pl.BlockSpec((4, SL, B, LN), lambda i: (0, 0, 0, 0))],
        out_specs=pl.BlockSpec(
            (FPI, B, K), lambda i: (jnp.maximum(i - 1, 0), 0, 0)),
        out_shape=jax.ShapeDtypeStruct((NPOINT, B, K), jnp.int32),
        scratch_shapes=[
            pltpu.VMEM((SL, B, LN), jnp.float32),
            pltpu.VMEM((3, B, 1), jnp.float32),
            pltpu.VMEM((2, FPI, SL, B, LN), jnp.float32),
        ],
    )(xyzj)


def _sc_gather(table, idx):
    info = plsc.get_sparse_core_info()
    nc, ns = info.num_cores, info.num_subcores
    nw = nc * ns
    total = B * NPOINT * K
    ch = total // (nw * 128)

    mesh = plsc.VectorSubcoreMesh(core_axis_name="c", subcore_axis_name="s")

    @functools.partial(
        pl.kernel,
        mesh=mesh,
        out_type=jax.ShapeDtypeStruct((nw, ch, 128, FPAD), jnp.float32),
        scratch_types=[
            pltpu.VMEM((ch, 128), jnp.int32),
            pltpu.VMEM((ch, 128, FPAD), jnp.float32),
            pltpu.SemaphoreType.DMA,
        ],
        compiler_params=pltpu.CompilerParams(use_tc_tiling_on_sc=False),
    )
    def gather_k(tab_hbm, idx_hbm, out_hbm, idx_v, rows_v, sem):
        w = lax.axis_index("s") * nc + lax.axis_index("c")
        pltpu.sync_copy(idx_hbm.at[w], idx_v)
        copies = []
        for c in range(ch):
            copies.append(
                pltpu.async_copy(tab_hbm.at[idx_v.at[c]], rows_v.at[c], sem))
        for cp in copies:
            cp.wait()
        pltpu.sync_copy(rows_v, out_hbm.at[w])

    return gather_k(table, idx)


def kernel(x):
    xyzT = (x[:, :, :3].transpose(2, 0, 1).reshape(3, B, SL, LN)
            .transpose(0, 2, 1, 3))
    knn = _fps_knn(xyzT)

    info = plsc.get_sparse_core_info()
    nw = info.num_cores * info.num_subcores
    total = B * NPOINT * K
    ch = total // (nw * 128)
    idx = knn.transpose(1, 0, 2).reshape(nw, ch, 128)

    table = jnp.pad(x, ((0, 0), (0, 0), (0, FPAD - FEAT))).reshape(B * N, FPAD)
    rows = _sc_gather(table, idx)
    return rows.reshape(B, NPOINT, K, FPAD)[..., :FEAT]
